# baseline (speedup 1.0000x reference)
import functools

import jax
import jax.numpy as jnp
from jax import lax
from jax.experimental import pallas as pl
from jax.experimental.pallas import tpu as pltpu
from jax.experimental.pallas import tpu_sc as plsc

N = 10000
E = 160000
IN_DIM = 256
HID_DIM = 512
OUT_DIM = 256

NC = 2
NS = 16
LANES = 16
HALF = 128
G = 128
NB = 80
E_PAD = NS * NB * G
NVEC = NB * G // LANES

NPASS = 4
RING = 2
GEN = RING * G
CAPB = 10496
NBLKB = CAPB // G
ACC_ROWS = 3328
TRASH = 1608
RPSA = ACC_ROWS // NS

ND_ACC = 10112
NBD = E_PAD // (NC * NS * G)
RPSD = ND_ACC // NS

RB = 400
GRID = N // RB

_mesh = plsc.VectorSubcoreMesh(core_axis_name="c", subcore_axis_name="s")


def _const_fill(buf, rows, cols, value):
    vec = jnp.full((LANES,), value, jnp.float32)

    @pl.loop(0, rows)
    def _(r):
        @pl.loop(0, cols // LANES)
        def _(c):
            buf[r, pl.ds(c * LANES, LANES)] = vec


@functools.partial(
    pl.kernel,
    mesh=_mesh,
    out_type=jax.ShapeDtypeStruct((NC * ND_ACC, HALF), jnp.float32),
    scratch_types=[
        pltpu.VMEM((NBD, G), jnp.int32),
        pltpu.VMEM((G, HALF), jnp.float32),
        pltpu.VMEM_SHARED((ND_ACC, HALF), jnp.float32),
    ],
)
def _sc_degree(dst_hbm, out_hbm, idx_v, ones_v, acc_sh):
    cid = lax.axis_index("c")
    sid = lax.axis_index("s")

    pltpu.sync_copy(dst_hbm.at[cid * NS + sid], idx_v)

    _const_fill(ones_v, G, HALF, 0.0)
    r0 = sid * RPSD

    @pl.loop(0, 4)
    def _(i):
        pltpu.sync_copy(ones_v, acc_sh.at[pl.ds(r0 + i * G, G)])

    pltpu.sync_copy(ones_v.at[pl.ds(0, RPSD - 4 * G)],
                    acc_sh.at[pl.ds(r0 + 4 * G, RPSD - 4 * G)])

    _const_fill(ones_v, G, HALF, 1.0)

    plsc.subcore_barrier()

    @pl.loop(0, NBD)
    def _(j):
        pltpu.sync_copy(ones_v, acc_sh.at[idx_v.at[j]], add=True)

    plsc.subcore_barrier()

    pltpu.sync_copy(acc_sh.at[pl.ds(r0, RPSD)],
                    out_hbm.at[pl.ds(cid * ND_ACC + r0, RPSD)])


@functools.partial(
    pl.kernel,
    mesh=_mesh,
    out_type=[
        jax.ShapeDtypeStruct((NC * NS * NPASS, CAPB), jnp.int32),
        jax.ShapeDtypeStruct((NC * NS * NPASS, 2 * CAPB), jnp.int32),
        jax.ShapeDtypeStruct((NC * NS, NPASS, HALF), jnp.int32),
    ],
    scratch_types=[
        pltpu.VMEM((NB, G), jnp.int32),
        pltpu.VMEM((NB, G), jnp.int32),
        pltpu.VMEM((CAPB,), jnp.int32),
        pltpu.VMEM((2 * CAPB,), jnp.int32),
        pltpu.VMEM((NPASS, HALF), jnp.int32),
    ],
    compiler_params=pltpu.CompilerParams(needs_layout_passes=False),
)
def _sc_partition(src_hbm, dst_hbm, srco_hbm, dsto_hbm, cnt_hbm,
                  in_src, in_dst, out_src, out_dst, cnt_v):
    cid = lax.axis_index("c")
    sid = lax.axis_index("s")
    w = cid * NS + sid

    pltpu.sync_copy(src_hbm.at[sid], in_src)
    pltpu.sync_copy(dst_hbm.at[sid], in_dst)

    qs = [cid * 5200 + p * 1600 for p in range(NPASS)]
    qe = [qs[p] + (1600 if p < 3 else 400 * (1 - cid)) for p in range(NPASS)]

    iot = lax.iota(jnp.int32, LANES)
    pair_idx = iot >> 1
    parity = iot & 1
    trash_src = jnp.zeros((LANES,), jnp.int32)
    trash_dst = jnp.full((LANES,), 2 * TRASH, jnp.int32) + parity

    for p in range(NPASS):
        def body(k, ptr, p=p):
            r = k // (G // LANES)
            c0 = (k % (G // LANES)) * LANES
            sv = in_src[r, pl.ds(c0, LANES)]
            dv = in_dst[r, pl.ds(c0, LANES)]
            rvec = jnp.full((LANES,), 1, jnp.int32) * r
            dpa = plsc.load_gather(in_dst, [rvec, c0 + pair_idx])
            dpb = plsc.load_gather(in_dst, [rvec, c0 + 8 + pair_idx])
            m = (dv >= qs[p]) & (dv < qe[p])
            plsc.store_compressed(out_src.at[pl.ds(ptr, LANES)], sv, mask=m)
            dptr = 2 * ptr
            ma = (dpa >= qs[p]) & (dpa < qe[p])
            da = (dpa - qs[p]) * 2 + parity
            plsc.store_compressed(out_dst.at[pl.ds(dptr, LANES)],
                                  da, mask=ma)
            na = jnp.max(plsc.all_reduce_population_count(ma))
            mb = (dpb >= qs[p]) & (dpb < qe[p])
            db = (dpb - qs[p]) * 2 + parity
            plsc.store_compressed(out_dst.at[pl.ds(dptr + na, LANES)],
                                  db, mask=mb)
            nb = jnp.max(plsc.all_reduce_population_count(mb))
            return ptr + (na + nb) // 2

        ptr = lax.fori_loop(0, NVEC, body, jnp.int32(0))

        for k in range(GEN // LANES + 1):
            out_src[pl.ds(ptr + k * LANES, LANES)] = trash_src
        for k in range(2 * (GEN // LANES) + 2):
            out_dst[pl.ds(2 * ptr + k * LANES, LANES)] = trash_dst
        ngen = (ptr + (GEN - lax.rem(ptr, GEN))) // GEN
        ngen_vec = jnp.full((LANES,), 1, jnp.int32) * ngen
        for k in range(HALF // LANES):
            cnt_v[p, pl.ds(k * LANES, LANES)] = ngen_vec

        pltpu.sync_copy(out_src, srco_hbm.at[w * NPASS + p])
        pltpu.sync_copy(out_dst, dsto_hbm.at[w * NPASS + p])

    pltpu.sync_copy(cnt_v, cnt_hbm.at[w])


@functools.partial(
    pl.kernel,
    mesh=_mesh,
    out_type=jax.ShapeDtypeStruct((NC * NPASS * ACC_ROWS, HALF), jnp.float32),
    scratch_types=[
        pltpu.VMEM((NBLKB, G), jnp.int32),
        pltpu.VMEM((2 * NBLKB, G), jnp.int32),
        pltpu.VMEM((NPASS, HALF), jnp.int32),
        pltpu.VMEM((RING, G, 2, HALF), jnp.float32),
        pltpu.VMEM_SHARED((ACC_ROWS, HALF), jnp.float32),
    ] + [pltpu.SemaphoreType.DMA] * (2 * RING),
)
def _sc_aggregate(srcb_hbm, dstb_hbm, cnt_hbm, table_hbm, out_hbm,
                  src_v, dst_v, cnt_v, bufs, acc_sh, *sems):
    gsems = sems[:RING]
    ssems = sems[RING:]
    cid = lax.axis_index("c")
    sid = lax.axis_index("s")
    w = cid * NS + sid
    r0 = sid * RPSA

    pltpu.sync_copy(cnt_hbm.at[w], cnt_v)

    for p in range(NPASS):
        pltpu.sync_copy(srcb_hbm.at[w, p], src_v)
        pltpu.sync_copy(dstb_hbm.at[w, p], dst_v)

        zv = bufs.at[0].reshape(2 * G, HALF)
        _const_fill(zv, 2 * G, HALF, 0.0)
        pltpu.sync_copy(zv.at[pl.ds(0, RPSA)], acc_sh.at[pl.ds(r0, RPSA)])

        ngen = cnt_v[p, pl.ds(0, LANES)][0]

        plsc.subcore_barrier()

        for b in range(RING):
            pltpu.async_copy(table_hbm.at[src_v.at[b]], bufs.at[b], gsems[b])

        @pl.loop(0, NBLKB // RING)
        def _(i):
            @pl.when(i < ngen)
            def _():
                j0 = i * RING
                for b in range(RING):
                    pltpu.make_async_copy(table_hbm.at[src_v.at[0]],
                                          bufs.at[b], gsems[b]).wait()
                    bview = bufs.at[b].reshape(2 * G, HALF)
                    pltpu.async_copy(bview.at[pl.ds(0, G)],
                                     acc_sh.at[dst_v.at[2 * (j0 + b)]],
                                     ssems[b], add=True)
                    pltpu.async_copy(bview.at[pl.ds(G, G)],
                                     acc_sh.at[dst_v.at[2 * (j0 + b) + 1]],
                                     ssems[b], add=True)
                for b in range(RING):
                    zz = bufs.at[b].reshape(2 * G, HALF)
                    pltpu.make_async_copy(zz.at[pl.ds(0, G)],
                                          acc_sh.at[dst_v.at[0]],
                                          ssems[b]).wait()
                    pltpu.make_async_copy(zz.at[pl.ds(G, G)],
                                          acc_sh.at[dst_v.at[1]],
                                          ssems[b]).wait()

                    @pl.when(i < ngen - 1)
                    def _():
                        pltpu.async_copy(
                            table_hbm.at[src_v.at[j0 + RING + b]],
                            bufs.at[b], gsems[b])

        plsc.subcore_barrier()

        pltpu.sync_copy(
            acc_sh.at[pl.ds(r0, RPSA)],
            out_hbm.at[pl.ds((cid * NPASS + p) * ACC_ROWS + r0, RPSA)])

        plsc.subcore_barrier()


def _dinv_of(deg_ref):
    return lax.rsqrt(deg_ref[0][:, :1] + deg_ref[1][:, :1] + 1.0)


def _scale(deg_ref, x_ref, o_ref):
    o_ref[...] = x_ref[...] * _dinv_of(deg_ref)


def _mm_chain(agg_ref, xp_ref, deg_ref, w1_ref, b1_ref, w2_ref,
              h_ref, zp_ref):
    dinv = _dinv_of(deg_ref)
    y = (agg_ref[0] + xp_ref[...]) * dinv
    x1 = jnp.dot(y, w1_ref[...], preferred_element_type=jnp.float32)
    h = jnp.maximum(x1 + b1_ref[...], 0.0)
    h_ref[...] = h
    z = jnp.dot(h, w2_ref[...], preferred_element_type=jnp.float32)
    zp_ref[...] = z * dinv


def _merge_bias(agg_ref, zp_ref, deg_ref, b2_ref, o_ref):
    dinv = _dinv_of(deg_ref)
    o_ref[...] = (agg_ref[0] + zp_ref[...]) * dinv + b2_ref[...]


def _agg_spec():
    def imap(i):
        i = jnp.asarray(i, jnp.int32)
        s = ((i >= 4).astype(jnp.int32) + (i >= 8).astype(jnp.int32)
             + (i >= 12).astype(jnp.int32) + (i >= 13).astype(jnp.int32)
             + (i >= 17).astype(jnp.int32) + (i >= 21).astype(jnp.int32))
        return (s, i - 4 * s + 3 * (s >= 4).astype(jnp.int32), 0)

    return pl.BlockSpec((1, RB, IN_DIM), imap)


def kernel(x, edge_index, W1, b1, W2, b2):
    src = edge_index[0]
    dst = edge_index[1]
    pad = E_PAD - E
    srcp = jnp.concatenate([src, jnp.zeros((pad,), jnp.int32)]).reshape(NS, NB, G)
    dstp = jnp.concatenate([dst, jnp.full((pad,), N, jnp.int32)]).reshape(NS, NB, G)

    deg2 = _sc_degree(dstp.reshape(NC * NS, NBD, G)).reshape(NC, ND_ACC, HALF)
    srcb, dstb, cnts = _sc_partition(srcp, dstp)
    srcb = srcb.reshape(NC * NS, NPASS, NBLKB, G)
    dstb = dstb.reshape(NC * NS, NPASS, 2 * NBLKB, G)

    xp = pl.pallas_call(
        _scale,
        grid=(GRID,),
        in_specs=[pl.BlockSpec((2, RB, HALF), lambda i: (0, i, 0)),
                  pl.BlockSpec((RB, IN_DIM), lambda i: (i, 0))],
        out_specs=pl.BlockSpec((RB, IN_DIM), lambda i: (i, 0)),
        out_shape=jax.ShapeDtypeStruct((N, IN_DIM), jnp.float32),
    )(deg2, x)

    agg1 = _sc_aggregate(srcb, dstb, cnts, xp.reshape(N, 2, HALF))
    agg1 = agg1.reshape(NC * NPASS, ACC_ROWS // 2, IN_DIM)

    h, zp = pl.pallas_call(
        _mm_chain,
        grid=(GRID,),
        in_specs=[_agg_spec(),
                  pl.BlockSpec((RB, IN_DIM), lambda i: (i, 0)),
                  pl.BlockSpec((2, RB, HALF), lambda i: (0, i, 0)),
                  pl.BlockSpec((IN_DIM, HID_DIM), lambda i: (0, 0)),
                  pl.BlockSpec((1, HID_DIM), lambda i: (0, 0)),
                  pl.BlockSpec((HID_DIM, OUT_DIM), lambda i: (0, 0))],
        out_specs=[pl.BlockSpec((RB, HID_DIM), lambda i: (i, 0)),
                   pl.BlockSpec((RB, OUT_DIM), lambda i: (i, 0))],
        out_shape=[jax.ShapeDtypeStruct((N, HID_DIM), jnp.float32),
                   jax.ShapeDtypeStruct((N, OUT_DIM), jnp.float32)],
    )(agg1, xp, deg2, W1, b1.reshape(1, HID_DIM), W2)

    agg2 = _sc_aggregate(srcb, dstb, cnts, zp.reshape(N, 2, HALF))
    agg2 = agg2.reshape(NC * NPASS, ACC_ROWS // 2, IN_DIM)

    x2 = pl.pallas_call(
        _merge_bias,
        grid=(GRID,),
        in_specs=[_agg_spec(),
                  pl.BlockSpec((RB, OUT_DIM), lambda i: (i, 0)),
                  pl.BlockSpec((2, RB, HALF), lambda i: (0, i, 0)),
                  pl.BlockSpec((1, OUT_DIM), lambda i: (0, 0))],
        out_specs=pl.BlockSpec((RB, OUT_DIM), lambda i: (i, 0)),
        out_shape=jax.ShapeDtypeStruct((N, OUT_DIM), jnp.float32),
    )(agg2, zp, deg2, b2.reshape(1, OUT_DIM))

    return (x2, h)


# speedup vs baseline: 2.6711x; 2.6711x over previous
import functools

import jax
import jax.numpy as jnp
from jax import lax
from jax.experimental import pallas as pl
from jax.experimental.pallas import tpu as pltpu
from jax.experimental.pallas import tpu_sc as plsc

N = 10000
E = 160000
IN_DIM = 256
HID_DIM = 512
OUT_DIM = 256

NC = 2
NS = 16
LANES = 16
HALF = 128
G = 128
NB = 80
CH = 16
NCHUNK = NB // CH
E_PAD = NS * NB * G
N_ACC = 10112
ROWS_PER_SUB = N_ACC // NS
RB = 400
GRID = N // RB

_mesh = plsc.VectorSubcoreMesh(core_axis_name="c", subcore_axis_name="s")


def _const_fill(buf, rows, cols, value):
    vec = jnp.full((LANES,), value, jnp.float32)

    @pl.loop(0, rows)
    def _(r):
        @pl.loop(0, cols // LANES)
        def _(c):
            buf[r, pl.ds(c * LANES, LANES)] = vec


def _zero_fill(buf, rows, cols):
    _const_fill(buf, rows, cols, 0.0)


NBD = E_PAD // (NC * NS * G)


@functools.partial(
    pl.kernel,
    mesh=_mesh,
    out_type=jax.ShapeDtypeStruct((NC * N_ACC, HALF), jnp.float32),
    scratch_types=[
        pltpu.VMEM((NBD, G), jnp.int32),
        pltpu.VMEM((G, HALF), jnp.float32),
        pltpu.VMEM_SHARED((N_ACC, HALF), jnp.float32),
    ],
)
def _sc_degree(dst_hbm, out_hbm, idx_v, ones_v, acc_sh):
    cid = lax.axis_index("c")
    sid = lax.axis_index("s")

    pltpu.sync_copy(dst_hbm.at[cid * NS + sid], idx_v)

    _zero_fill(ones_v, G, HALF)
    r0 = sid * ROWS_PER_SUB

    @pl.loop(0, 4)
    def _(i):
        pltpu.sync_copy(ones_v, acc_sh.at[pl.ds(r0 + i * G, G)])

    pltpu.sync_copy(ones_v.at[pl.ds(0, ROWS_PER_SUB - 4 * G)],
                    acc_sh.at[pl.ds(r0 + 4 * G, ROWS_PER_SUB - 4 * G)])

    _const_fill(ones_v, G, HALF, 1.0)

    plsc.subcore_barrier()

    @pl.loop(0, NBD)
    def _(j):
        pltpu.sync_copy(ones_v, acc_sh.at[idx_v.at[j]], add=True)

    plsc.subcore_barrier()

    pltpu.sync_copy(acc_sh.at[pl.ds(r0, ROWS_PER_SUB)],
                    out_hbm.at[pl.ds(cid * N_ACC + r0, ROWS_PER_SUB)])


@functools.partial(
    pl.kernel,
    mesh=_mesh,
    out_type=jax.ShapeDtypeStruct((NC * N_ACC, HALF), jnp.float32),
    scratch_types=[
        pltpu.VMEM((CH, G), jnp.int32),
        pltpu.VMEM((CH, G), jnp.int32),
        pltpu.VMEM((G, HALF), jnp.float32),
        pltpu.VMEM((G, HALF), jnp.float32),
        pltpu.VMEM_SHARED((N_ACC, HALF), jnp.float32),
        pltpu.SemaphoreType.DMA,
        pltpu.SemaphoreType.DMA,
    ],
)
def _sc_aggregate(src_hbm, dst_hbm, table_hbm, out_hbm,
                  src_v, dst_v, buf0, buf1, acc_sh, gsem0, gsem1):
    cid = lax.axis_index("c")
    sid = lax.axis_index("s")

    _zero_fill(buf0, G, HALF)
    r0 = sid * ROWS_PER_SUB

    @pl.loop(0, 4)
    def _(i):
        pltpu.sync_copy(buf0, acc_sh.at[pl.ds(r0 + i * G, G)])

    pltpu.sync_copy(buf0.at[pl.ds(0, ROWS_PER_SUB - 4 * G)],
                    acc_sh.at[pl.ds(r0 + 4 * G, ROWS_PER_SUB - 4 * G)])

    off = cid * N
    plsc.subcore_barrier()

    @pl.loop(0, NCHUNK)
    def _(q):
        pltpu.sync_copy(src_hbm.at[sid, pl.ds(q * CH, CH)], src_v)
        pltpu.sync_copy(dst_hbm.at[sid, pl.ds(q * CH, CH)], dst_v)

        @pl.loop(0, CH)
        def _(j):
            @pl.loop(0, G // LANES)
            def _(c):
                src_v[j, pl.ds(c * LANES, LANES)] = (
                    src_v[j, pl.ds(c * LANES, LANES)] + off)

        pltpu.async_copy(table_hbm.at[src_v.at[0]], buf0, gsem0).wait()

        @pl.loop(0, CH // 2 - 1)
        def _(i):
            j = i * 2
            cp1 = pltpu.async_copy(table_hbm.at[src_v.at[j + 1]], buf1, gsem1)
            pltpu.sync_copy(buf0, acc_sh.at[dst_v.at[j]], add=True)
            cp1.wait()
            cp0 = pltpu.async_copy(table_hbm.at[src_v.at[j + 2]], buf0, gsem0)
            pltpu.sync_copy(buf1, acc_sh.at[dst_v.at[j + 1]], add=True)
            cp0.wait()

        cp1 = pltpu.async_copy(table_hbm.at[src_v.at[CH - 1]], buf1, gsem1)
        pltpu.sync_copy(buf0, acc_sh.at[dst_v.at[CH - 2]], add=True)
        cp1.wait()
        pltpu.sync_copy(buf1, acc_sh.at[dst_v.at[CH - 1]], add=True)

    plsc.subcore_barrier()

    pltpu.sync_copy(acc_sh.at[pl.ds(r0, ROWS_PER_SUB)],
                    out_hbm.at[pl.ds(cid * N_ACC + r0, ROWS_PER_SUB)])


def _dinv_of(deg_ref):
    return lax.rsqrt(deg_ref[0][:, :1] + deg_ref[1][:, :1] + 1.0)


def _scale_split(deg_ref, x_ref, o_ref):
    xp = x_ref[...] * _dinv_of(deg_ref)
    o_ref[0] = xp[:, :HALF]
    o_ref[1] = xp[:, HALF:]


def _mm_chain(agg_ref, xp_ref, deg_ref, w1_ref, b1_ref, w2_ref,
              h_ref, zp_ref):
    dinv = _dinv_of(deg_ref)
    y = jnp.concatenate([(agg_ref[0] + xp_ref[0]) * dinv,
                         (agg_ref[1] + xp_ref[1]) * dinv], axis=1)
    x1 = jnp.dot(y, w1_ref[...], preferred_element_type=jnp.float32)
    h = jnp.maximum(x1 + b1_ref[...], 0.0)
    h_ref[...] = h
    z = jnp.dot(h, w2_ref[...], preferred_element_type=jnp.float32)
    zp = z * dinv
    zp_ref[0] = zp[:, :HALF]
    zp_ref[1] = zp[:, HALF:]


def _merge_bias(agg_ref, zp_ref, deg_ref, b2_ref, o_ref):
    dinv = _dinv_of(deg_ref)
    o_ref[...] = jnp.concatenate([(agg_ref[0] + zp_ref[0]) * dinv,
                                  (agg_ref[1] + zp_ref[1]) * dinv],
                                 axis=1) + b2_ref[...]


def kernel(x, edge_index, W1, b1, W2, b2):
    src = edge_index[0]
    dst = edge_index[1]
    pad = E_PAD - E
    srcp = jnp.concatenate([src, jnp.zeros((pad,), jnp.int32)]).reshape(NS, NB, G)
    dstp = jnp.concatenate([dst, jnp.full((pad,), N, jnp.int32)]).reshape(NS, NB, G)

    deg2 = _sc_degree(dstp.reshape(NC * NS, NBD, G)).reshape(NC, N_ACC, HALF)

    xp_st = pl.pallas_call(
        _scale_split,
        grid=(GRID,),
        in_specs=[pl.BlockSpec((2, RB, HALF), lambda i: (0, i, 0)),
                  pl.BlockSpec((RB, IN_DIM), lambda i: (i, 0))],
        out_specs=pl.BlockSpec((2, RB, HALF), lambda i: (0, i, 0)),
        out_shape=jax.ShapeDtypeStruct((2, N, HALF), jnp.float32),
    )(deg2, x)

    agg1 = _sc_aggregate(srcp, dstp, xp_st.reshape(2 * N, HALF))
    agg1 = agg1.reshape(2, N_ACC, HALF)

    h, zp_st = pl.pallas_call(
        _mm_chain,
        grid=(GRID,),
        in_specs=[pl.BlockSpec((2, RB, HALF), lambda i: (0, i, 0)),
                  pl.BlockSpec((2, RB, HALF), lambda i: (0, i, 0)),
                  pl.BlockSpec((2, RB, HALF), lambda i: (0, i, 0)),
                  pl.BlockSpec((IN_DIM, HID_DIM), lambda i: (0, 0)),
                  pl.BlockSpec((1, HID_DIM), lambda i: (0, 0)),
                  pl.BlockSpec((HID_DIM, OUT_DIM), lambda i: (0, 0))],
        out_specs=[pl.BlockSpec((RB, HID_DIM), lambda i: (i, 0)),
                   pl.BlockSpec((2, RB, HALF), lambda i: (0, i, 0))],
        out_shape=[jax.ShapeDtypeStruct((N, HID_DIM), jnp.float32),
                   jax.ShapeDtypeStruct((2, N, HALF), jnp.float32)],
    )(agg1, xp_st, deg2, W1, b1.reshape(1, HID_DIM), W2)

    agg2 = _sc_aggregate(srcp, dstp, zp_st.reshape(2 * N, HALF))
    agg2 = agg2.reshape(2, N_ACC, HALF)

    x2 = pl.pallas_call(
        _merge_bias,
        grid=(GRID,),
        in_specs=[pl.BlockSpec((2, RB, HALF), lambda i: (0, i, 0)),
                  pl.BlockSpec((2, RB, HALF), lambda i: (0, i, 0)),
                  pl.BlockSpec((2, RB, HALF), lambda i: (0, i, 0)),
                  pl.BlockSpec((1, OUT_DIM), lambda i: (0, 0))],
        out_specs=pl.BlockSpec((RB, OUT_DIM), lambda i: (i, 0)),
        out_shape=jax.ShapeDtypeStruct((N, OUT_DIM), jnp.float32),
    )(agg2, zp_st, deg2, b2.reshape(1, OUT_DIM))

    return (x2, h)
